# baseline (speedup 1.0000x reference)
import functools
import math

import jax
import jax.numpy as jnp
from jax import lax
from jax.experimental import pallas as pl
from jax.experimental.pallas import tpu as pltpu
from jax.experimental.pallas import tpu_sc as plsc

NUM_CODEBOOKS = 4
VOCAB = 100000
D = 128
B, S = 1024, 200
N = B * S
SCALE = math.sqrt(D)

NC, NS, L = 2, 16, 16
NW = NC * NS
B_PER_W = N // NW
CHUNK = 128
NCHUNK = B_PER_W // CHUNK

TBLK = 5000


def _transform_tables(w0, w1, w2, w3, comb_w, comb_b2d):
    def body(w0_ref, w1_ref, w2_ref, w3_ref, cw_ref, cb_ref,
             t0_ref, t1_ref, t2_ref, t3_ref):
        c = cw_ref[...]
        for i, (w_ref, t_ref) in enumerate(
                zip((w0_ref, w1_ref, w2_ref, w3_ref),
                    (t0_ref, t1_ref, t2_ref, t3_ref))):
            acc = jnp.dot(w_ref[...], c[i * D:(i + 1) * D, :],
                          preferred_element_type=jnp.float32) * SCALE
            if i == 0:
                acc = acc + cb_ref[...] * SCALE
            t_ref[...] = acc

    tbl_spec = pl.BlockSpec((TBLK, D), lambda r: (r, 0))
    return pl.pallas_call(
        body,
        grid=(VOCAB // TBLK,),
        in_specs=[tbl_spec] * 4 + [
            pl.BlockSpec((NUM_CODEBOOKS * D, D), lambda r: (0, 0)),
            pl.BlockSpec((1, D), lambda r: (0, 0)),
        ],
        out_specs=[tbl_spec] * 4,
        out_shape=[jax.ShapeDtypeStruct((VOCAB, D), jnp.float32)] * 4,
    )(w0, w1, w2, w3, comb_w, comb_b2d)


def _gather_sum(idx_flat, t0, t1, t2, t3):
    mesh = plsc.VectorSubcoreMesh(core_axis_name="c", subcore_axis_name="s")

    scratch = (
        [pltpu.VMEM((B_PER_W,), jnp.int32)] * 4
        + [pltpu.VMEM((CHUNK, D), jnp.float32)] * 2
        + [pltpu.SemaphoreType.DMA] * 2
        + [pltpu.SemaphoreType.DMA] * 2
        + [pltpu.SemaphoreType.DMA] * 2
    )

    @functools.partial(
        pl.kernel,
        mesh=mesh,
        out_type=jax.ShapeDtypeStruct((N, D), jnp.float32),
        scratch_types=scratch,
    )
    def k(idx_hbm, t0_hbm, t1_hbm, t2_hbm, t3_hbm, out_hbm,
          ix0, ix1, ix2, ix3, ba, bb,
          ga, gb, aa, ab, oa, ob):
        wid = lax.axis_index("s") * NC + lax.axis_index("c")
        cbase = wid * NCHUNK
        ixs = (ix0, ix1, ix2, ix3)
        tbls = (t0_hbm, t1_hbm, t2_hbm, t3_hbm)
        bufs = (ba, bb)
        gsems = (ga, gb)
        asems = (aa, ab)
        osems = (oa, ob)
        for i in range(NUM_CODEBOOKS):
            pltpu.sync_copy(
                idx_hbm.at[pl.ds(i * N + wid * B_PER_W, B_PER_W)], ixs[i])

        def base_cp(ch, p):
            return pltpu.make_async_copy(
                t0_hbm.at[ixs[0].at[pl.ds(ch * CHUNK, CHUNK)]],
                bufs[p], gsems[p])

        def add_src(ch, i):
            return tbls[i].at[ixs[i].at[pl.ds(ch * CHUNK, CHUNK)]]

        def add_wait_cp(ch, p, i):
            return pltpu.make_async_copy(add_src(ch, i), bufs[p], asems[p])

        def out_cp(ch, p):
            return pltpu.make_async_copy(
                bufs[p], out_hbm.at[pl.ds((cbase + ch) * CHUNK, CHUNK)],
                osems[p])

        base_cp(0, 0).start()

        def pair_body(j, _):
            for p in range(2):
                q = 1 - p
                ch = 2 * j + p
                base_cp(ch, p).wait()
                for i in range(1, NUM_CODEBOOKS):
                    pltpu.async_copy(add_src(ch, i), bufs[p], asems[p],
                                     add=True)

                @pl.when(j + p >= 1)
                def _():
                    out_cp(ch, q).wait()

                if p == 0:
                    base_cp(ch + 1, q).start()
                else:
                    @pl.when(j < NCHUNK // 2 - 1)
                    def _():
                        base_cp(ch + 1, q).start()
                for i in range(1, NUM_CODEBOOKS):
                    add_wait_cp(ch, p, i).wait()
                out_cp(ch, p).start()
            return 0

        lax.fori_loop(0, NCHUNK // 2, pair_body, 0, unroll=False)
        out_cp(NCHUNK - 1, 1).wait()

    return k(idx_flat, t0, t1, t2, t3)


def kernel(tokens, W0, W1, W2, W3, comb_W, comb_b):
    t0, t1, t2, t3 = _transform_tables(
        W0, W1, W2, W3, comb_W, comb_b.reshape(1, D))
    idx_flat = (
        tokens.astype(jnp.int32)
        .reshape(N, NUM_CODEBOOKS)
        .T.reshape(NUM_CODEBOOKS * N)
    )
    out = _gather_sum(idx_flat, t0, t1, t2, t3)
    return out.reshape(B, S, D)
